# baseline (speedup 1.0000x reference)
import functools

import jax
import jax.numpy as jnp
from jax import lax
from jax.experimental import pallas as pl
from jax.experimental.pallas import tpu as pltpu
from jax.experimental.pallas import tpu_sc as plsc

E = 8
TOP_K = 2
D = 1024
DFF = 768
S = 2048

BLK = 256
NUM_BLOCKS = (S * TOP_K) // BLK + E
P = NUM_BLOCKS * BLK

NC = 2
NS = 16
NW = NC * NS

TOK_PER_W = S // NW
DCH = 16
DNCH = TOK_PER_W // DCH
CCH = 16
ROWS_PER_W = P // NW


def _router_body(x_ref, gw_ref, logits_ref, top_ref, w_ref):
    x = x_ref[...]
    gw = gw_ref[...]
    logits = lax.dot_general(x, gw, (((1,), (1,)), ((), ())),
                             preferred_element_type=jnp.float32)
    logits_ref[...] = logits
    iota = lax.broadcasted_iota(jnp.int32, logits.shape, 1)
    m1 = jnp.max(logits, axis=1, keepdims=True)
    a1 = jnp.min(jnp.where(logits == m1, iota, jnp.int32(E)), axis=1,
                 keepdims=True)
    masked = jnp.where(iota == a1, -jnp.inf, logits)
    m2 = jnp.max(masked, axis=1, keepdims=True)
    a2 = jnp.min(jnp.where(masked == m2, iota, jnp.int32(E)), axis=1,
                 keepdims=True)
    w1 = jax.nn.sigmoid(m1 - m2)
    top_ref[...] = jnp.concatenate([a1, a2], axis=1)
    w_ref[...] = jnp.concatenate([w1, 1.0 - w1], axis=1)


def _run_router(x, gate_w):
    return pl.pallas_call(
        _router_body,
        out_shape=(
            jax.ShapeDtypeStruct((S, E), jnp.float32),
            jax.ShapeDtypeStruct((S, TOP_K), jnp.int32),
            jax.ShapeDtypeStruct((S, TOP_K), jnp.float32),
        ),
    )(x, gate_w)


def _expert_body(be_ref, ba_ref, xs_ref, gp_ref, up_ref, dp_ref, ws_ref,
                 ys_ref, gpb_ref, upb_ref, dpb_ref):
    b = pl.program_id(0)
    prev = be_ref[jnp.maximum(b - 1, 0)]
    changed = jnp.logical_or(b == 0, be_ref[b] != prev)

    @pl.when(jnp.logical_and(ba_ref[b] == 1, changed))
    def _():
        gpb_ref[...] = gp_ref[0].astype(jnp.bfloat16)
        upb_ref[...] = up_ref[0].astype(jnp.bfloat16)
        dpb_ref[...] = dp_ref[0].astype(jnp.bfloat16)

    @pl.when(ba_ref[b] == 1)
    def _():
        x = xs_ref[...]
        g = lax.dot_general(x, gpb_ref[...], (((1,), (1,)), ((), ())),
                            preferred_element_type=jnp.float32)
        u = lax.dot_general(x, upb_ref[...], (((1,), (1,)), ((), ())),
                            preferred_element_type=jnp.float32)
        h = ((g * jax.nn.sigmoid(g)) * u).astype(jnp.bfloat16)
        y = lax.dot_general(h, dpb_ref[...], (((1,), (1,)), ((), ())),
                            preferred_element_type=jnp.float32)
        ys_ref[...] = y * ws_ref[...]

    @pl.when(ba_ref[b] != 1)
    def _():
        ys_ref[...] = jnp.zeros_like(ys_ref)


def _run_experts(block_expert, block_active, xs, gate_proj, up_proj,
                 down_proj, w_pos):
    grid_spec = pltpu.PrefetchScalarGridSpec(
        num_scalar_prefetch=2,
        grid=(NUM_BLOCKS,),
        in_specs=[
            pl.BlockSpec((BLK, D), lambda b, be, ba: (b, 0)),
            pl.BlockSpec((1, DFF, D), lambda b, be, ba: (be[b], 0, 0)),
            pl.BlockSpec((1, DFF, D), lambda b, be, ba: (be[b], 0, 0)),
            pl.BlockSpec((1, D, DFF), lambda b, be, ba: (be[b], 0, 0)),
            pl.BlockSpec((BLK, 1), lambda b, be, ba: (b, 0)),
        ],
        out_specs=pl.BlockSpec((BLK, D), lambda b, be, ba: (b, 0)),
        scratch_shapes=[
            pltpu.VMEM((DFF, D), jnp.bfloat16),
            pltpu.VMEM((DFF, D), jnp.bfloat16),
            pltpu.VMEM((D, DFF), jnp.bfloat16),
        ],
    )
    return pl.pallas_call(
        _expert_body,
        grid_spec=grid_spec,
        out_shape=jax.ShapeDtypeStruct((P, D), jnp.float32),
    )(block_expert, block_active, xs, gate_proj, up_proj, down_proj, w_pos)


@functools.cache
def _sc_kernels():
    mesh = plsc.VectorSubcoreMesh(core_axis_name="c", subcore_axis_name="s")

    @functools.partial(
        pl.kernel,
        out_type=jax.ShapeDtypeStruct((P, D // 2), jnp.int32),
        mesh=mesh,
        scratch_types=(
            [pltpu.VMEM((DNCH, DCH), jnp.int32)] * 2
            + [pltpu.VMEM((DCH, D // 2), jnp.int32)] * DNCH
            + [pltpu.SemaphoreType.DMA] * (3 * DNCH)
        ),
    )
    def dispatch_scatter(x_hbm, pos0_hbm, pos1_hbm, out_hbm, p0_v, p1_v,
                         *bufs_sems):
        bufs = bufs_sems[:DNCH]
        sems = bufs_sems[DNCH:]
        wid = lax.axis_index("s") * NC + lax.axis_index("c")
        tbase = wid * TOK_PER_W
        pltpu.sync_copy(pos0_hbm.at[wid], p0_v)
        pltpu.sync_copy(pos1_hbm.at[wid], p1_v)
        gcp = [
            pltpu.async_copy(
                x_hbm.at[pl.ds(tbase + c * DCH, DCH)], bufs[c],
                sems[3 * c])
            for c in range(DNCH)
        ]
        scp = []
        for c in range(DNCH):
            gcp[c].wait()
            scp.append(pltpu.async_copy(
                bufs[c], out_hbm.at[p0_v.at[c]], sems[3 * c + 1]))
            scp.append(pltpu.async_copy(
                bufs[c], out_hbm.at[p1_v.at[c]], sems[3 * c + 2]))
        for cp in scp:
            cp.wait()

    @functools.partial(
        pl.kernel,
        out_type=jax.ShapeDtypeStruct((S, D), jnp.float32),
        mesh=mesh,
        scratch_types=[
            pltpu.VMEM((TOK_PER_W,), jnp.int32),
            pltpu.VMEM((TOK_PER_W,), jnp.int32),
            pltpu.VMEM((CCH, D), jnp.float32),
            pltpu.VMEM((CCH, D), jnp.float32),
            pltpu.VMEM((CCH, D), jnp.float32),
            pltpu.VMEM((CCH, D), jnp.float32),
            pltpu.SemaphoreType.DMA,
            pltpu.SemaphoreType.DMA,
            pltpu.SemaphoreType.DMA,
            pltpu.SemaphoreType.DMA,
            pltpu.SemaphoreType.DMA,
            pltpu.SemaphoreType.DMA,
        ],
    )
    def combine(ys_hbm, pos0_hbm, pos1_hbm, out_hbm, i0_v, i1_v,
                a0_v, a1_v, b0_v, b1_v, s0a, s0b, s1a, s1b, os0, os1):
        wid = lax.axis_index("s") * NC + lax.axis_index("c")
        base = wid * TOK_PER_W
        pltpu.sync_copy(pos0_hbm.at[pl.ds(base, TOK_PER_W)], i0_v)
        pltpu.sync_copy(pos1_hbm.at[pl.ds(base, TOK_PER_W)], i1_v)
        nch = TOK_PER_W // CCH
        r0s = [a0_v, b0_v]
        r1s = [a1_v, b1_v]
        g0sems = [s0a, s1a]
        g1sems = [s0b, s1b]
        osems = [os0, os1]

        def start_gathers(c, buf):
            c0 = pltpu.async_copy(
                ys_hbm.at[i0_v.at[pl.ds(c * CCH, CCH)]], r0s[buf],
                g0sems[buf])
            c1 = pltpu.async_copy(
                ys_hbm.at[i1_v.at[pl.ds(c * CCH, CCH)]], r1s[buf],
                g1sems[buf])
            return c0, c1

        gcp = [None] * nch
        ocp = [None] * nch
        gcp[0] = start_gathers(0, 0)
        for c in range(nch):
            cur = c % 2
            nxt = (c + 1) % 2
            if c + 1 < nch:
                if ocp[c - 1] is not None:
                    ocp[c - 1].wait()
                gcp[c + 1] = start_gathers(c + 1, nxt)
            gcp[c][0].wait()
            gcp[c][1].wait()

            def row_add(r, carry):
                for j in range(D // 16):
                    sl = pl.ds(j * 16, 16)
                    r0s[cur][r, sl] = r0s[cur][r, sl] + r1s[cur][r, sl]
                return carry

            lax.fori_loop(0, CCH, row_add, 0)
            ocp[c] = pltpu.async_copy(
                r0s[cur], out_hbm.at[pl.ds(base + c * CCH, CCH)],
                osems[cur])
        ocp[nch - 2].wait()
        ocp[nch - 1].wait()

    return dispatch_scatter, combine


def _build_plan(top):
    sel = top.reshape(-1)
    oh = (sel[:, None] == jnp.arange(E, dtype=jnp.int32)[None, :]
          ).astype(jnp.int32)
    csum = jnp.cumsum(oh, axis=0)
    rank = jnp.take_along_axis(csum - 1, sel[:, None], axis=1)[:, 0]
    counts = csum[-1]
    used = (counts + BLK - 1) // BLK
    cum_used = jnp.cumsum(used)
    pad_start = (cum_used - used) * BLK
    pos = pad_start[sel] + rank
    total_used = cum_used[-1]
    bidx = jnp.arange(NUM_BLOCKS, dtype=jnp.int32)
    be_raw = jnp.sum(
        (bidx[:, None] >= cum_used[None, :]).astype(jnp.int32), axis=1)
    be_last = be_raw[jnp.maximum(total_used - 1, 0)]
    block_expert = jnp.where(bidx < total_used, be_raw, be_last)
    block_active = (bidx < total_used).astype(jnp.int32)
    pos0 = pos[0::TOP_K]
    pos1 = pos[1::TOP_K]
    return block_expert, block_active, pos, pos0, pos1


def kernel(hidden_states, gate_w, gate_proj, up_proj, down_proj):
    b, s, d = hidden_states.shape
    x = hidden_states.reshape(s, d)
    router_logits, top, w = _run_router(x, gate_w)
    block_expert, block_active, pos, pos0, pos1 = _build_plan(top)
    dispatch_scatter, combine = _sc_kernels()
    xb = lax.bitcast_convert_type(
        x.astype(jnp.bfloat16).reshape(S, D // 2, 2), jnp.int32)
    xs_i = dispatch_scatter(
        xb,
        pos0.reshape(NW, DNCH, DCH),
        pos1.reshape(NW, DNCH, DCH),
    )
    xs = lax.bitcast_convert_type(xs_i, jnp.bfloat16).reshape(P, D)
    w_pos = jnp.zeros((P,), jnp.float32).at[pos].set(
        w.reshape(-1), unique_indices=True, mode="promise_in_bounds")
    ys = _run_experts(block_expert, block_active, xs, gate_proj, up_proj,
                      down_proj, w_pos.reshape(P, 1))
    out = combine(ys, pos0, pos1)
    return out.reshape(b, s, d), router_logits


# speedup vs baseline: 2.1493x; 2.1493x over previous
import functools

import jax
import jax.numpy as jnp
from jax import lax
from jax.experimental import pallas as pl
from jax.experimental.pallas import tpu as pltpu
from jax.experimental.pallas import tpu_sc as plsc

E = 8
TOP_K = 2
D = 1024
DFF = 768
S = 2048

BLK = 256
NUM_BLOCKS = (S * TOP_K) // BLK + E
P = NUM_BLOCKS * BLK

NC = 2
NS = 16
NW = NC * NS

TOK_PER_W = S // NW
DCH = 16
DNCH = TOK_PER_W // DCH
CCH = 16
ROWS_PER_W = P // NW


def _router_body(x_ref, gw_ref, logits_ref, top_ref, w_ref, xb_ref):
    x = x_ref[...]
    gw = gw_ref[...]
    logits = lax.dot_general(x, gw, (((1,), (1,)), ((), ())),
                             preferred_element_type=jnp.float32)
    logits_ref[...] = logits
    bb = lax.bitcast_convert_type(
        x.astype(jnp.bfloat16).astype(jnp.float32), jnp.int32) >> 16
    xb_ref[...] = (bb[:, : D // 2] & 0xFFFF) | (bb[:, D // 2:] << 16)
    iota = lax.broadcasted_iota(jnp.int32, logits.shape, 1)
    m1 = jnp.max(logits, axis=1, keepdims=True)
    a1 = jnp.min(jnp.where(logits == m1, iota, jnp.int32(E)), axis=1,
                 keepdims=True)
    masked = jnp.where(iota == a1, -jnp.inf, logits)
    m2 = jnp.max(masked, axis=1, keepdims=True)
    a2 = jnp.min(jnp.where(masked == m2, iota, jnp.int32(E)), axis=1,
                 keepdims=True)
    w1 = jax.nn.sigmoid(m1 - m2)
    top_ref[...] = jnp.concatenate([a1, a2], axis=1)
    w_ref[...] = jnp.concatenate([w1, 1.0 - w1], axis=1)


def _run_router(x, gate_w):
    return pl.pallas_call(
        _router_body,
        out_shape=(
            jax.ShapeDtypeStruct((S, E), jnp.float32),
            jax.ShapeDtypeStruct((S, TOP_K), jnp.int32),
            jax.ShapeDtypeStruct((S, TOP_K), jnp.float32),
            jax.ShapeDtypeStruct((S, D // 2), jnp.int32),
        ),
    )(x, gate_w)


def _expert_body(be_ref, ba_ref, xs_ref, gp_ref, up_ref, dp_ref, ws_ref,
                 ys_ref, gpb_ref, upb_ref, dpb_ref):
    b = pl.program_id(0)
    prev = be_ref[jnp.maximum(b - 1, 0)]
    changed = jnp.logical_or(b == 0, be_ref[b] != prev)

    @pl.when(jnp.logical_and(ba_ref[b] == 1, changed))
    def _():
        gpb_ref[...] = gp_ref[0].astype(jnp.bfloat16)
        upb_ref[...] = up_ref[0].astype(jnp.bfloat16)
        dpb_ref[...] = dp_ref[0].astype(jnp.bfloat16)

    @pl.when(ba_ref[b] == 1)
    def _():
        xi = xs_ref[...]
        fl = lax.bitcast_convert_type(xi << 16, jnp.float32)
        fh = lax.bitcast_convert_type(xi & jnp.int32(-65536), jnp.float32)
        x = jnp.concatenate(
            [fl.astype(jnp.bfloat16), fh.astype(jnp.bfloat16)], axis=1)
        g = lax.dot_general(x, gpb_ref[...], (((1,), (1,)), ((), ())),
                            preferred_element_type=jnp.float32)
        u = lax.dot_general(x, upb_ref[...], (((1,), (1,)), ((), ())),
                            preferred_element_type=jnp.float32)
        h = ((g * jax.nn.sigmoid(g)) * u).astype(jnp.bfloat16)
        y = lax.dot_general(h, dpb_ref[...], (((1,), (1,)), ((), ())),
                            preferred_element_type=jnp.float32)
        ys_ref[...] = y * ws_ref[...]

    @pl.when(ba_ref[b] != 1)
    def _():
        ys_ref[...] = jnp.zeros_like(ys_ref)


def _run_experts(block_expert, block_active, xs, gate_proj, up_proj,
                 down_proj, w_pos):
    grid_spec = pltpu.PrefetchScalarGridSpec(
        num_scalar_prefetch=2,
        grid=(NUM_BLOCKS,),
        in_specs=[
            pl.BlockSpec((BLK, D // 2), lambda b, be, ba: (b, 0)),
            pl.BlockSpec((1, DFF, D), lambda b, be, ba: (be[b], 0, 0)),
            pl.BlockSpec((1, DFF, D), lambda b, be, ba: (be[b], 0, 0)),
            pl.BlockSpec((1, D, DFF), lambda b, be, ba: (be[b], 0, 0)),
            pl.BlockSpec((BLK, 1), lambda b, be, ba: (b, 0)),
        ],
        out_specs=pl.BlockSpec((BLK, D), lambda b, be, ba: (b, 0)),
        scratch_shapes=[
            pltpu.VMEM((DFF, D), jnp.bfloat16),
            pltpu.VMEM((DFF, D), jnp.bfloat16),
            pltpu.VMEM((D, DFF), jnp.bfloat16),
        ],
    )
    return pl.pallas_call(
        _expert_body,
        grid_spec=grid_spec,
        out_shape=jax.ShapeDtypeStruct((P, D), jnp.float32),
    )(block_expert, block_active, xs, gate_proj, up_proj, down_proj, w_pos)


@functools.cache
def _sc_kernels():
    mesh = plsc.VectorSubcoreMesh(core_axis_name="c", subcore_axis_name="s")

    @functools.partial(
        pl.kernel,
        out_type=jax.ShapeDtypeStruct((P, D // 2), jnp.int32),
        mesh=mesh,
        scratch_types=(
            [pltpu.VMEM((DNCH, DCH), jnp.int32)] * 2
            + [pltpu.VMEM((DCH, D // 2), jnp.int32)] * DNCH
            + [pltpu.SemaphoreType.DMA] * (3 * DNCH)
        ),
    )
    def dispatch_scatter(x_hbm, pos0_hbm, pos1_hbm, out_hbm, p0_v, p1_v,
                         *bufs_sems):
        bufs = bufs_sems[:DNCH]
        sems = bufs_sems[DNCH:]
        wid = lax.axis_index("s") * NC + lax.axis_index("c")
        tbase = wid * TOK_PER_W
        pltpu.sync_copy(pos0_hbm.at[wid], p0_v)
        pltpu.sync_copy(pos1_hbm.at[wid], p1_v)
        gcp = [
            pltpu.async_copy(
                x_hbm.at[pl.ds(tbase + c * DCH, DCH)], bufs[c],
                sems[3 * c])
            for c in range(DNCH)
        ]
        scp = []
        for c in range(DNCH):
            gcp[c].wait()
            scp.append(pltpu.async_copy(
                bufs[c], out_hbm.at[p0_v.at[c]], sems[3 * c + 1]))
            scp.append(pltpu.async_copy(
                bufs[c], out_hbm.at[p1_v.at[c]], sems[3 * c + 2]))
        for cp in scp:
            cp.wait()

    @functools.partial(
        pl.kernel,
        out_type=jax.ShapeDtypeStruct((S, D), jnp.float32),
        mesh=mesh,
        scratch_types=[
            pltpu.VMEM((TOK_PER_W,), jnp.int32),
            pltpu.VMEM((TOK_PER_W,), jnp.int32),
            pltpu.VMEM((CCH, D), jnp.float32),
            pltpu.VMEM((CCH, D), jnp.float32),
            pltpu.VMEM((CCH, D), jnp.float32),
            pltpu.VMEM((CCH, D), jnp.float32),
            pltpu.SemaphoreType.DMA,
            pltpu.SemaphoreType.DMA,
            pltpu.SemaphoreType.DMA,
            pltpu.SemaphoreType.DMA,
            pltpu.SemaphoreType.DMA,
            pltpu.SemaphoreType.DMA,
        ],
    )
    def combine(ys_hbm, pos0_hbm, pos1_hbm, out_hbm, i0_v, i1_v,
                a0_v, a1_v, b0_v, b1_v, s0a, s0b, s1a, s1b, os0, os1):
        wid = lax.axis_index("s") * NC + lax.axis_index("c")
        base = wid * TOK_PER_W
        pltpu.sync_copy(pos0_hbm.at[pl.ds(base, TOK_PER_W)], i0_v)
        pltpu.sync_copy(pos1_hbm.at[pl.ds(base, TOK_PER_W)], i1_v)
        nch = TOK_PER_W // CCH
        r0s = [a0_v, b0_v]
        r1s = [a1_v, b1_v]
        g0sems = [s0a, s1a]
        g1sems = [s0b, s1b]
        osems = [os0, os1]

        def start_gathers(c, buf):
            c0 = pltpu.async_copy(
                ys_hbm.at[i0_v.at[pl.ds(c * CCH, CCH)]], r0s[buf],
                g0sems[buf])
            c1 = pltpu.async_copy(
                ys_hbm.at[i1_v.at[pl.ds(c * CCH, CCH)]], r1s[buf],
                g1sems[buf])
            return c0, c1

        gcp = [None] * nch
        ocp = [None] * nch
        gcp[0] = start_gathers(0, 0)
        for c in range(nch):
            cur = c % 2
            nxt = (c + 1) % 2
            if c + 1 < nch:
                if ocp[c - 1] is not None:
                    ocp[c - 1].wait()
                gcp[c + 1] = start_gathers(c + 1, nxt)
            gcp[c][0].wait()
            gcp[c][1].wait()

            def row_add(r, carry):
                for j in range(D // 16):
                    sl = pl.ds(j * 16, 16)
                    r0s[cur][r, sl] = r0s[cur][r, sl] + r1s[cur][r, sl]
                return carry

            lax.fori_loop(0, CCH, row_add, 0)
            ocp[c] = pltpu.async_copy(
                r0s[cur], out_hbm.at[pl.ds(base + c * CCH, CCH)],
                osems[cur])
        ocp[nch - 2].wait()
        ocp[nch - 1].wait()

    return dispatch_scatter, combine


def _build_plan(top):
    sel = top.reshape(-1)
    oh = (sel[:, None] == jnp.arange(E, dtype=jnp.int32)[None, :]
          ).astype(jnp.int32)
    csum = jnp.cumsum(oh, axis=0)
    rank = jnp.take_along_axis(csum - 1, sel[:, None], axis=1)[:, 0]
    counts = csum[-1]
    used = (counts + BLK - 1) // BLK
    cum_used = jnp.cumsum(used)
    pad_start = (cum_used - used) * BLK
    pos = pad_start[sel] + rank
    total_used = cum_used[-1]
    bidx = jnp.arange(NUM_BLOCKS, dtype=jnp.int32)
    be_raw = jnp.sum(
        (bidx[:, None] >= cum_used[None, :]).astype(jnp.int32), axis=1)
    be_last = be_raw[jnp.maximum(total_used - 1, 0)]
    block_expert = jnp.where(bidx < total_used, be_raw, be_last)
    block_active = (bidx < total_used).astype(jnp.int32)
    pos0 = pos[0::TOP_K]
    pos1 = pos[1::TOP_K]
    return block_expert, block_active, pos, pos0, pos1


def kernel(hidden_states, gate_w, gate_proj, up_proj, down_proj):
    b, s, d = hidden_states.shape
    x = hidden_states.reshape(s, d)
    router_logits, top, w, xb = _run_router(x, gate_w)
    block_expert, block_active, pos, pos0, pos1 = _build_plan(top)
    dispatch_scatter, combine = _sc_kernels()
    xs = dispatch_scatter(
        xb,
        pos0.reshape(NW, DNCH, DCH),
        pos1.reshape(NW, DNCH, DCH),
    )
    w_pos = jnp.zeros((P,), jnp.float32).at[pos].set(
        w.reshape(-1), unique_indices=True, mode="promise_in_bounds")
    ys = _run_experts(block_expert, block_active, xs, gate_proj, up_proj,
                      down_proj, w_pos.reshape(P, 1))
    out = combine(ys, pos0, pos1)
    return out.reshape(b, s, d), router_logits
